# Initial kernel scaffold; baseline (speedup 1.0000x reference)
import jax
import jax.numpy as jnp
from jax.experimental import pallas as pl


def kernel(full_board_vector, piece_ids, proj_w, proj_b):
    raise NotImplementedError("write your pallas kernel here")


# speedup vs baseline: 26.8274x; 26.8274x over previous
import functools
import jax
import jax.numpy as jnp
from jax.experimental import pallas as pl
from jax.experimental.pallas import tpu as pltpu

B, C, HW, P, OUT = 16384, 11, 64, 32, 64
BLK = 128


def _tc_body(ids_ref, board_ref, wt_ref, bias_ref, out_ref):
    ids = ids_ref[...]
    board = board_ref[...]
    wt = wt_ref[...]
    bias = bias_ref[...]

    targets = jax.lax.broadcasted_iota(jnp.int32, (1, P, 1), 1) + 1
    mask = (ids[:, None, :] == targets)
    maskf = mask.astype(jnp.float32)

    row = jax.lax.broadcasted_iota(jnp.int32, (HW, HW), 0)
    col = jax.lax.broadcasted_iota(jnp.int32, (HW, HW), 1)
    tri = (row <= col).astype(jnp.float32)
    cum = jnp.dot(maskf.reshape(BLK * P, HW), tri,
                  preferred_element_type=jnp.float32)
    first = (cum == 1.0).reshape(BLK, P, HW)
    e = jnp.where(mask & first, 1.0, 0.0)

    raw = jax.lax.dot_general(
        e, board,
        dimension_numbers=(((2,), (2,)), ((0,), (0,))),
        preferred_element_type=jnp.float32)

    out = jnp.dot(raw.reshape(BLK * P, C), wt,
                  preferred_element_type=jnp.float32) + bias
    out_ref[...] = out.reshape(BLK, P, OUT)


@jax.jit
def kernel(full_board_vector, piece_ids, proj_w, proj_b):
    ids = piece_ids.reshape(B, HW)
    board = full_board_vector.reshape(B, C, HW)
    wt = proj_w.T
    bias = proj_b.reshape(1, OUT)

    grid = (B // BLK,)
    out = pl.pallas_call(
        _tc_body,
        grid=grid,
        in_specs=[
            pl.BlockSpec((BLK, HW), lambda i: (i, 0)),
            pl.BlockSpec((BLK, C, HW), lambda i: (i, 0, 0)),
            pl.BlockSpec((C, OUT), lambda i: (0, 0)),
            pl.BlockSpec((1, OUT), lambda i: (0, 0)),
        ],
        out_specs=pl.BlockSpec((BLK, P, OUT), lambda i: (i, 0, 0)),
        out_shape=jax.ShapeDtypeStruct((B, P, OUT), jnp.float32),
    )(ids, board, wt, bias)
    return out
